# baseline (speedup 1.0000x reference)
import jax
import jax.numpy as jnp
from jax.experimental import pallas as pl
from jax.experimental.pallas import tpu as pltpu

_SUB = 128
_CHUNK = 2048


def _body(x_ref, o_ref, carry_ref):
    j = pl.program_id(0)

    @pl.when(j == 0)
    def _init():
        carry_ref[...] = jnp.zeros_like(carry_ref)

    nsub = _CHUNK // _SUB

    rows = jax.lax.broadcasted_iota(jnp.int32, (_SUB, _SUB), 0)
    cols = jax.lax.broadcasted_iota(jnp.int32, (_SUB, _SUB), 1)
    tri = (rows <= cols).astype(jnp.float32)

    c1 = jax.lax.broadcasted_iota(jnp.int32, (_CHUNK, nsub), 0)
    k1 = jax.lax.broadcasted_iota(jnp.int32, (_CHUNK, nsub), 1)
    emat = (c1 // _SUB == k1).astype(jnp.float32)

    k2 = jax.lax.broadcasted_iota(jnp.int32, (nsub, _CHUNK), 0)
    c2 = jax.lax.broadcasted_iota(jnp.int32, (nsub, _CHUNK), 1)
    tmat = (k2 < c2 // _SUB).astype(jnp.float32)

    x = x_ref[...]
    ends = jnp.dot(x, emat, preferred_element_type=jnp.float32)
    offs = jnp.dot(ends, tmat, preferred_element_type=jnp.float32)
    base = carry_ref[...]
    for k in range(nsub):
        xk = x[:, k * _SUB:(k + 1) * _SUB]
        sk = jnp.dot(xk, tri, preferred_element_type=jnp.float32)
        o_ref[:, k * _SUB:(k + 1) * _SUB] = (
            sk + offs[:, k * _SUB:(k + 1) * _SUB] + base
        )
    carry_ref[...] = base + jnp.sum(ends, axis=1, keepdims=True)


def kernel(x):
    m, n = x.shape
    grid = (n // _CHUNK,)
    return pl.pallas_call(
        _body,
        grid=grid,
        in_specs=[pl.BlockSpec((m, _CHUNK), lambda j: (0, j))],
        out_specs=pl.BlockSpec((m, _CHUNK), lambda j: (0, j)),
        out_shape=jax.ShapeDtypeStruct((m, n), jnp.float32),
        scratch_shapes=[pltpu.VMEM((m, 1), jnp.float32)],
    )(x)


# speedup vs baseline: 1.4473x; 1.4473x over previous
import jax
import jax.numpy as jnp
from jax.experimental import pallas as pl
from jax.experimental.pallas import tpu as pltpu

_SUB = 128
_CHUNK = 8192


def _body(x_ref, o_ref, carry_ref):
    j = pl.program_id(0)

    @pl.when(j == 0)
    def _init():
        carry_ref[...] = jnp.zeros_like(carry_ref)

    nsub = _CHUNK // _SUB

    rows = jax.lax.broadcasted_iota(jnp.int32, (_SUB, _SUB), 0)
    cols = jax.lax.broadcasted_iota(jnp.int32, (_SUB, _SUB), 1)
    tri = (rows <= cols).astype(jnp.float32)

    c1 = jax.lax.broadcasted_iota(jnp.int32, (_CHUNK, nsub), 0)
    k1 = jax.lax.broadcasted_iota(jnp.int32, (_CHUNK, nsub), 1)
    emat = (c1 // _SUB == k1).astype(jnp.float32)

    k2 = jax.lax.broadcasted_iota(jnp.int32, (nsub, _CHUNK), 0)
    c2 = jax.lax.broadcasted_iota(jnp.int32, (nsub, _CHUNK), 1)
    tmat = (k2 < c2 // _SUB).astype(jnp.float32)

    x = x_ref[...]
    ends = jnp.dot(x, emat, preferred_element_type=jnp.float32)
    offs = jnp.dot(ends, tmat, preferred_element_type=jnp.float32)
    base = carry_ref[...]
    for k in range(nsub):
        xk = x[:, k * _SUB:(k + 1) * _SUB]
        sk = jnp.dot(xk, tri, preferred_element_type=jnp.float32)
        o_ref[:, k * _SUB:(k + 1) * _SUB] = (
            sk + offs[:, k * _SUB:(k + 1) * _SUB] + base
        )
    carry_ref[...] = base + jnp.sum(ends, axis=1, keepdims=True)


def kernel(x):
    m, n = x.shape
    grid = (n // _CHUNK,)
    return pl.pallas_call(
        _body,
        grid=grid,
        in_specs=[pl.BlockSpec((m, _CHUNK), lambda j: (0, j))],
        out_specs=pl.BlockSpec((m, _CHUNK), lambda j: (0, j)),
        out_shape=jax.ShapeDtypeStruct((m, n), jnp.float32),
        scratch_shapes=[pltpu.VMEM((m, 1), jnp.float32)],
    )(x)


# speedup vs baseline: 1.4566x; 1.0064x over previous
import jax
import jax.numpy as jnp
from jax.experimental import pallas as pl
from jax.experimental.pallas import tpu as pltpu

_SUB = 128
_CHUNK = 16384


def _body(x_ref, o_ref, carry_ref):
    j = pl.program_id(0)

    @pl.when(j == 0)
    def _init():
        carry_ref[...] = jnp.zeros_like(carry_ref)

    nsub = _CHUNK // _SUB

    rows = jax.lax.broadcasted_iota(jnp.int32, (_SUB, _SUB), 0)
    cols = jax.lax.broadcasted_iota(jnp.int32, (_SUB, _SUB), 1)
    tri = (rows <= cols).astype(jnp.float32)

    c1 = jax.lax.broadcasted_iota(jnp.int32, (_CHUNK, nsub), 0)
    k1 = jax.lax.broadcasted_iota(jnp.int32, (_CHUNK, nsub), 1)
    emat = (c1 // _SUB == k1).astype(jnp.float32)

    k2 = jax.lax.broadcasted_iota(jnp.int32, (nsub, _CHUNK), 0)
    c2 = jax.lax.broadcasted_iota(jnp.int32, (nsub, _CHUNK), 1)
    tmat = (k2 < c2 // _SUB).astype(jnp.float32)

    x = x_ref[...]
    ends = jnp.dot(x, emat, preferred_element_type=jnp.float32)
    offs = jnp.dot(ends, tmat, preferred_element_type=jnp.float32)
    base = carry_ref[...]
    for k in range(nsub):
        xk = x[:, k * _SUB:(k + 1) * _SUB]
        sk = jnp.dot(xk, tri, preferred_element_type=jnp.float32)
        o_ref[:, k * _SUB:(k + 1) * _SUB] = (
            sk + offs[:, k * _SUB:(k + 1) * _SUB] + base
        )
    carry_ref[...] = base + jnp.sum(ends, axis=1, keepdims=True)


def kernel(x):
    m, n = x.shape
    grid = (n // _CHUNK,)
    return pl.pallas_call(
        _body,
        grid=grid,
        in_specs=[pl.BlockSpec((m, _CHUNK), lambda j: (0, j))],
        out_specs=pl.BlockSpec((m, _CHUNK), lambda j: (0, j)),
        out_shape=jax.ShapeDtypeStruct((m, n), jnp.float32),
        scratch_shapes=[pltpu.VMEM((m, 1), jnp.float32)],
    )(x)
